# baseline (speedup 1.0000x reference)
import jax
import jax.numpy as jnp
from jax import lax
from jax.experimental import pallas as pl
from jax.experimental.pallas import tpu as pltpu

NUM_TOKENS = 8192
NUM_EXPERTS = 64
CAPACITY = 160
BLK = 128


def _router_kernel(x_ref, combine_ref, aoh_ref, boh_ref, cnt_ref):
    step = pl.program_id(0)

    @pl.when(step == 0)
    def _():
        cnt_ref[...] = jnp.zeros_like(cnt_ref)

    x = x_ref[...]
    m = jnp.max(x, axis=0, keepdims=True)
    s = jnp.sum(jnp.exp(x - m), axis=0, keepdims=True)
    w_row = 1.0 / s

    e_iota = lax.broadcasted_iota(jnp.int32, (NUM_EXPERTS, BLK), 0)
    cand = jnp.where(x == m, e_iota, NUM_EXPERTS)
    e_row = jnp.min(cand, axis=0, keepdims=True)

    oh_msk = e_iota == e_row
    oh = oh_msk.astype(jnp.float32)

    r_iota = lax.broadcasted_iota(jnp.int32, (BLK, BLK), 0)
    c_iota = lax.broadcasted_iota(jnp.int32, (BLK, BLK), 1)
    triu = (r_iota < c_iota).astype(jnp.float32)
    ranks_excl = jnp.dot(oh, triu, preferred_element_type=jnp.float32)

    r_all = cnt_ref[...] + ranks_excl
    r_row = jnp.sum(oh * r_all, axis=0, keepdims=True)
    cnt_ref[...] = cnt_ref[...] + jnp.sum(oh, axis=1, keepdims=True)

    a_mat = oh * w_row
    cap_iota = lax.broadcasted_iota(jnp.int32, (CAPACITY, BLK), 0)
    b_msk = cap_iota == r_row.astype(jnp.int32)
    b_mat = b_msk.astype(jnp.float32)

    combine_ref[...] = a_mat[:, None, :] * b_mat[None, :, :]
    aoh_ref[...] = oh_msk.astype(jnp.int8)
    boh_ref[...] = b_msk.astype(jnp.int8)


def kernel(inputs):
    grid = NUM_TOKENS // BLK
    x_t = inputs.astype(jnp.float32).T
    combine_t, aoh_t, boh_t = pl.pallas_call(
        _router_kernel,
        grid=(grid,),
        in_specs=[pl.BlockSpec((NUM_EXPERTS, BLK), lambda i: (0, i))],
        out_specs=[
            pl.BlockSpec((NUM_EXPERTS, CAPACITY, BLK), lambda i: (0, 0, i)),
            pl.BlockSpec((NUM_EXPERTS, BLK), lambda i: (0, i)),
            pl.BlockSpec((CAPACITY, BLK), lambda i: (0, i)),
        ],
        out_shape=[
            jax.ShapeDtypeStruct((NUM_EXPERTS, CAPACITY, NUM_TOKENS), jnp.float32),
            jax.ShapeDtypeStruct((NUM_EXPERTS, NUM_TOKENS), jnp.int8),
            jax.ShapeDtypeStruct((CAPACITY, NUM_TOKENS), jnp.int8),
        ],
        scratch_shapes=[pltpu.VMEM((NUM_EXPERTS, 1), jnp.float32)],
    )(x_t)
    combine = jnp.transpose(combine_t, (2, 0, 1))
    a_bool = jnp.transpose(aoh_t, (1, 0)).view(jnp.bool_)
    b_bool = jnp.transpose(boh_t, (1, 0)).view(jnp.bool_)
    sec = a_bool[:, :, None] & b_bool[:, None, :]
    return (combine, sec)


# speedup vs baseline: 1.0033x; 1.0033x over previous
import jax
import jax.numpy as jnp
from jax import lax
from jax.experimental import pallas as pl
from jax.experimental.pallas import tpu as pltpu

NUM_TOKENS = 8192
NUM_EXPERTS = 64
CAPACITY = 160
BLK = 256


def _router_kernel(x_ref, combine_ref, aoh_ref, boh_ref, cnt_ref):
    step = pl.program_id(0)

    @pl.when(step == 0)
    def _():
        cnt_ref[...] = jnp.zeros_like(cnt_ref)

    x = x_ref[...]
    m = jnp.max(x, axis=0, keepdims=True)
    s = jnp.sum(jnp.exp(x - m), axis=0, keepdims=True)
    w_row = 1.0 / s

    e_iota = lax.broadcasted_iota(jnp.int32, (NUM_EXPERTS, BLK), 0)
    cand = jnp.where(x == m, e_iota, NUM_EXPERTS)
    e_row = jnp.min(cand, axis=0, keepdims=True)

    oh_msk = e_iota == e_row
    oh = oh_msk.astype(jnp.float32)

    r_iota = lax.broadcasted_iota(jnp.int32, (BLK, BLK), 0)
    c_iota = lax.broadcasted_iota(jnp.int32, (BLK, BLK), 1)
    triu = (r_iota < c_iota).astype(jnp.float32)
    ranks_excl = jnp.dot(oh, triu, preferred_element_type=jnp.float32)

    r_all = cnt_ref[...] + ranks_excl
    r_row = jnp.sum(oh * r_all, axis=0, keepdims=True)
    cnt_ref[...] = cnt_ref[...] + jnp.sum(oh, axis=1, keepdims=True)

    a_mat = oh * w_row
    cap_iota = lax.broadcasted_iota(jnp.int32, (CAPACITY, BLK), 0)
    b_msk = cap_iota == r_row.astype(jnp.int32)
    b_mat = b_msk.astype(jnp.float32)

    combine_ref[...] = a_mat[:, None, :] * b_mat[None, :, :]
    aoh_ref[...] = oh_msk.astype(jnp.int8)
    boh_ref[...] = b_msk.astype(jnp.int8)


def kernel(inputs):
    grid = NUM_TOKENS // BLK
    x_t = inputs.astype(jnp.float32).T
    combine_t, aoh_t, boh_t = pl.pallas_call(
        _router_kernel,
        grid=(grid,),
        in_specs=[pl.BlockSpec((NUM_EXPERTS, BLK), lambda i: (0, i))],
        out_specs=[
            pl.BlockSpec((NUM_EXPERTS, CAPACITY, BLK), lambda i: (0, 0, i)),
            pl.BlockSpec((NUM_EXPERTS, BLK), lambda i: (0, i)),
            pl.BlockSpec((CAPACITY, BLK), lambda i: (0, i)),
        ],
        out_shape=[
            jax.ShapeDtypeStruct((NUM_EXPERTS, CAPACITY, NUM_TOKENS), jnp.float32),
            jax.ShapeDtypeStruct((NUM_EXPERTS, NUM_TOKENS), jnp.int8),
            jax.ShapeDtypeStruct((CAPACITY, NUM_TOKENS), jnp.int8),
        ],
        scratch_shapes=[pltpu.VMEM((NUM_EXPERTS, 1), jnp.float32)],
    )(x_t)
    combine = jnp.transpose(combine_t, (2, 0, 1))
    a_bool = jnp.transpose(aoh_t, (1, 0)).view(jnp.bool_)
    b_bool = jnp.transpose(boh_t, (1, 0)).view(jnp.bool_)
    sec = a_bool[:, :, None] & b_bool[:, None, :]
    return (combine, sec)
